# Initial kernel scaffold; baseline (speedup 1.0000x reference)
import jax
import jax.numpy as jnp
from jax import lax
from jax.experimental import pallas as pl
from jax.experimental.pallas import tpu as pltpu
from jax.experimental.pallas import tpu_sc as plsc

N = 100000
E = 3200000
EMB = 32
HID = 8
OUT = 16
K = 10
ALPHA = 0.1

NC = 2
NS = 16
NW = NC * NS

CH = 128
GRP = 8
EPAD = 3211264
EROWS = EPAD // CH
TROWS = EROWS // NW

ACC_R = N + 16
ZBLK = ACC_R // NS
RBLK = N // NS

DEG_R = 100096
DZBLK = DEG_R // NS

_mesh = plsc.VectorSubcoreMesh(core_axis_name="c", subcore_axis_name="s")

def _deg_body(dstp, zeros1, out, dacc, idx_v, ones_v):
    c = lax.axis_index("c")
    s = lax.axis_index("s")
    w = c * NS + s
    for i in range(CH // 16):
        ones_v[pl.ds(i * 16, 16)] = jnp.full((16,), 1.0, jnp.float32)
    pltpu.sync_copy(zeros1, dacc.at[pl.ds(s * DZBLK, DZBLK)])
    plsc.subcore_barrier()
    row0 = w * TROWS

    def grp(g, carry):
        pltpu.sync_copy(dstp.at[pl.ds(row0 + g * 16, 16)], idx_v)
        for j in range(16):
            pltpu.sync_copy(ones_v, dacc.at[idx_v.at[j]], add=True)
        return carry

    lax.fori_loop(0, TROWS // 16, grp, 0)
    plsc.subcore_barrier()
    pltpu.sync_copy(dacc.at[pl.ds(s * DZBLK, DZBLK)],
                    out.at[c].at[pl.ds(s * DZBLK, DZBLK)])


_deg_call = pl.kernel(
    _deg_body,
    out_type=jax.ShapeDtypeStruct((NC, DEG_R), jnp.float32),
    mesh=_mesh,
    scratch_types=[
        pltpu.VMEM_SHARED((DEG_R,), jnp.float32),
        pltpu.VMEM((16, CH), jnp.int32),
        pltpu.VMEM((CH,), jnp.float32),
    ],
)


def _scat_body(srcp, dstp, z, zeros2, out, acc, si, di, rows, sem):
    c = lax.axis_index("c")
    s = lax.axis_index("s")
    w = c * NS + s
    pltpu.sync_copy(zeros2, acc.at[pl.ds(s * ZBLK, ZBLK)])
    plsc.subcore_barrier()
    row0 = w * TROWS

    def grp(g, carry):
        base = row0 + g * GRP
        pltpu.sync_copy(srcp.at[pl.ds(base, GRP)], si)
        pltpu.sync_copy(dstp.at[pl.ds(base, GRP)], di)
        descs = [
            pltpu.async_copy(z.at[si.at[j]], rows.at[pl.ds(j * CH, CH)], sem)
            for j in range(GRP)
        ]
        for d in descs:
            d.wait()
        for j in range(GRP):
            pltpu.sync_copy(rows.at[pl.ds(j * CH, CH)], acc.at[di.at[j]],
                            add=True)
        return carry

    lax.fori_loop(0, TROWS // GRP, grp, 0)
    plsc.subcore_barrier()
    pltpu.sync_copy(acc.at[pl.ds(s * RBLK, RBLK)],
                    out.at[c].at[pl.ds(s * RBLK, RBLK)])


_scat_call = pl.kernel(
    _scat_body,
    out_type=jax.ShapeDtypeStruct((NC, N, OUT), jnp.float32),
    mesh=_mesh,
    scratch_types=[
        pltpu.VMEM_SHARED((ACC_R, OUT), jnp.float32),
        pltpu.VMEM((GRP, CH), jnp.int32),
        pltpu.VMEM((GRP, CH), jnp.int32),
        pltpu.VMEM((GRP * CH, OUT), jnp.float32),
        pltpu.SemaphoreType.DMA,
    ],
)


def _dot_body(srcp, dstp, x, out, si, di, ra, rb, pv, sem):
    c = lax.axis_index("c")
    s = lax.axis_index("s")
    w = c * NS + s
    row0 = w * TROWS

    def grp(g, carry):
        base = row0 + g * GRP
        pltpu.sync_copy(srcp.at[pl.ds(base, GRP)], si)
        pltpu.sync_copy(dstp.at[pl.ds(base, GRP)], di)
        da = [
            pltpu.async_copy(x.at[si.at[j]], ra.at[pl.ds(j * CH, CH)], sem)
            for j in range(GRP)
        ]
        db = [
            pltpu.async_copy(x.at[di.at[j]], rb.at[pl.ds(j * CH, CH)], sem)
            for j in range(GRP)
        ]
        for d in da:
            d.wait()
        for d in db:
            d.wait()

        def dot1(i, cc):
            pv[i] = jnp.sum(ra[i] * rb[i])
            return cc

        lax.fori_loop(0, GRP * CH, dot1, 0)
        pltpu.sync_copy(pv, out.at[pl.ds(base * CH, GRP * CH)])
        return carry

    lax.fori_loop(0, TROWS // GRP, grp, 0)


_dot_call = pl.kernel(
    _dot_body,
    out_type=jax.ShapeDtypeStruct((EPAD,), jnp.float32),
    mesh=_mesh,
    scratch_types=[
        pltpu.VMEM((GRP, CH), jnp.int32),
        pltpu.VMEM((GRP, CH), jnp.int32),
        pltpu.VMEM((GRP * CH, OUT), jnp.float32),
        pltpu.VMEM((GRP * CH, OUT), jnp.float32),
        pltpu.VMEM((GRP * CH,), jnp.float32),
        pltpu.SemaphoreType.DMA,
    ],
)


BN = 2000


def _prep_body(emb, W1, b1, W2, b2, d0, d1, z0_o, c_o, d2x_o, dfx_o, ch_o):
    x1 = jnp.maximum(
        jnp.dot(emb[...], W1[...], preferred_element_type=jnp.float32)
        + b1[...], 0.0)
    x0 = jnp.dot(x1, W2[...], preferred_element_type=jnp.float32) + b2[...]
    deg = d0[...] + d1[...] + 1.0
    dis = lax.rsqrt(deg)
    z0_o[...] = x0 * dis
    c_o[...] = ALPHA * dis * x0
    d2x_o[...] = jnp.broadcast_to((1.0 - ALPHA) * dis * dis, (BN, OUT))
    dfx_o[...] = jnp.broadcast_to((1.0 - ALPHA) * dis, (BN, OUT))
    ch_o[...] = ALPHA * x0


_prep_call = pl.pallas_call(
    _prep_body,
    grid=(N // BN,),
    in_specs=[
        pl.BlockSpec((BN, EMB), lambda i: (i, 0)),
        pl.BlockSpec((EMB, HID), lambda i: (0, 0)),
        pl.BlockSpec((1, HID), lambda i: (0, 0)),
        pl.BlockSpec((HID, OUT), lambda i: (0, 0)),
        pl.BlockSpec((1, OUT), lambda i: (0, 0)),
        pl.BlockSpec((BN, 1), lambda i: (i, 0)),
        pl.BlockSpec((BN, 1), lambda i: (i, 0)),
    ],
    out_specs=[pl.BlockSpec((BN, OUT), lambda i: (i, 0))] * 5,
    out_shape=[jax.ShapeDtypeStruct((N, OUT), jnp.float32)] * 5,
)


def _blend_body(s0, s1, z, coef, addt, o):
    o[...] = coef[...] * (s0[...] + s1[...] + z[...]) + addt[...]


_blend_call = pl.pallas_call(
    _blend_body,
    grid=(N // BN,),
    in_specs=[pl.BlockSpec((BN, OUT), lambda i: (i, 0))] * 5,
    out_specs=pl.BlockSpec((BN, OUT), lambda i: (i, 0)),
    out_shape=jax.ShapeDtypeStruct((N, OUT), jnp.float32),
)


def kernel(edge_index, emb, W1, b1, W2, b2):
    src = edge_index[0]
    dst = edge_index[1]
    pad = EPAD - E
    srcp = jnp.concatenate(
        [src, jnp.zeros((pad,), jnp.int32)]).reshape(EROWS, CH)
    dstp = jnp.concatenate(
        [dst, jnp.full((pad,), N, jnp.int32)]).reshape(EROWS, CH)
    dstp0 = jnp.concatenate(
        [dst, jnp.zeros((pad,), jnp.int32)]).reshape(EROWS, CH)

    zeros1 = jnp.zeros((DZBLK,), jnp.float32)
    zeros2 = jnp.zeros((ZBLK, OUT), jnp.float32)

    dparts = _deg_call(dstp, zeros1)
    d0 = dparts[0].reshape(DEG_R, 1)
    d1 = dparts[1].reshape(DEG_R, 1)

    z0, c, d2x, dfx, ch = _prep_call(
        emb, W1, b1.reshape(1, HID), W2, b2.reshape(1, OUT), d0, d1)

    z = z0
    for k in range(K):
        S = _scat_call(srcp, dstp, z, zeros2)
        if k < K - 1:
            z = _blend_call(S[0], S[1], z, d2x, c)
        else:
            x = _blend_call(S[0], S[1], z, dfx, ch)

    predp = _dot_call(srcp, dstp0, x)
    return predp[:E]


# speedup vs baseline: 25.3099x; 25.3099x over previous
import jax
import jax.numpy as jnp
from jax import lax
from jax.experimental import pallas as pl
from jax.experimental.pallas import tpu as pltpu
from jax.experimental.pallas import tpu_sc as plsc

N = 100000
E = 3200000
EMB = 32
HID = 8
OUT = 16
K = 10
ALPHA = 0.1

NC = 2
NS = 16
NW = NC * NS

CH = 128
GRP = 8
EPAD = 3211264
EROWS = EPAD // CH
TROWS = EROWS // NW

ACC_R = 100096
ZBLK = ACC_R // NS

DEG_R = 100352
DZBLK = DEG_R // NS

_mesh = plsc.VectorSubcoreMesh(core_axis_name="c", subcore_axis_name="s")

def _deg_body(dstp, zeros1, out, dacc, idx_v, ones_v):
    c = lax.axis_index("c")
    s = lax.axis_index("s")
    w = c * NS + s
    for i in range(CH // 16):
        ones_v[pl.ds(i * 16, 16)] = jnp.full((16,), 1.0, jnp.float32)
    pltpu.sync_copy(zeros1, dacc.at[pl.ds(s * DZBLK, DZBLK)])
    plsc.subcore_barrier()
    row0 = w * TROWS

    def grp(g, carry):
        pltpu.sync_copy(dstp.at[pl.ds(row0 + g * 16, 16)], idx_v)
        for j in range(16):
            pltpu.sync_copy(ones_v, dacc.at[idx_v.at[j]], add=True)
        return carry

    lax.fori_loop(0, TROWS // 16, grp, 0)
    plsc.subcore_barrier()
    pltpu.sync_copy(dacc.at[pl.ds(s * DZBLK, DZBLK)],
                    out.at[c].at[pl.ds(s * DZBLK, DZBLK)])


_deg_call = pl.kernel(
    _deg_body,
    out_type=jax.ShapeDtypeStruct((NC, DEG_R), jnp.float32),
    mesh=_mesh,
    compiler_params=pltpu.CompilerParams(use_tc_tiling_on_sc=False, needs_layout_passes=False),
    scratch_types=[
        pltpu.VMEM_SHARED((DEG_R,), jnp.float32),
        pltpu.VMEM((16, CH), jnp.int32),
        pltpu.VMEM((CH,), jnp.float32),
    ],
)


def _scat_body(srcp, dstp, z, zeros2, out, acc, si, di, rows, sem):
    c = lax.axis_index("c")
    s = lax.axis_index("s")
    w = c * NS + s
    pltpu.sync_copy(zeros2, acc.at[pl.ds(s * ZBLK, ZBLK)])
    plsc.subcore_barrier()
    row0 = w * TROWS

    def grp(g, carry):
        base = row0 + g * GRP
        pltpu.sync_copy(srcp.at[pl.ds(base, GRP)], si)
        pltpu.sync_copy(dstp.at[pl.ds(base, GRP)], di)
        descs = [
            pltpu.async_copy(z.at[si.at[j]], rows.at[pl.ds(j * CH, CH)], sem)
            for j in range(GRP)
        ]
        for d in descs:
            d.wait()
        for j in range(GRP):
            pltpu.sync_copy(rows.at[pl.ds(j * CH, CH)], acc.at[di.at[j]],
                            add=True)
        return carry

    lax.fori_loop(0, TROWS // GRP, grp, 0)
    plsc.subcore_barrier()
    pltpu.sync_copy(acc.at[pl.ds(s * ZBLK, ZBLK)],
                    out.at[c].at[pl.ds(s * ZBLK, ZBLK)])


_scat_call = pl.kernel(
    _scat_body,
    out_type=jax.ShapeDtypeStruct((NC, ACC_R, OUT), jnp.float32),
    mesh=_mesh,
    compiler_params=pltpu.CompilerParams(use_tc_tiling_on_sc=False, needs_layout_passes=False),
    scratch_types=[
        pltpu.VMEM_SHARED((ACC_R, OUT), jnp.float32),
        pltpu.VMEM((GRP, CH), jnp.int32),
        pltpu.VMEM((GRP, CH), jnp.int32),
        pltpu.VMEM((GRP * CH, OUT), jnp.float32),
        pltpu.SemaphoreType.DMA,
    ],
)


def _dot_body(srcp, dstp, x, out, si, di, ra, rb, pv, sem):
    c = lax.axis_index("c")
    s = lax.axis_index("s")
    w = c * NS + s
    row0 = w * TROWS

    def grp(g, carry):
        base = row0 + g * GRP
        pltpu.sync_copy(srcp.at[pl.ds(base, GRP)], si)
        pltpu.sync_copy(dstp.at[pl.ds(base, GRP)], di)
        da = [
            pltpu.async_copy(x.at[si.at[j]], ra.at[pl.ds(j * CH, CH)], sem)
            for j in range(GRP)
        ]
        db = [
            pltpu.async_copy(x.at[di.at[j]], rb.at[pl.ds(j * CH, CH)], sem)
            for j in range(GRP)
        ]
        for d in da:
            d.wait()
        for d in db:
            d.wait()

        def dot16(i, cc):
            e0 = i * 16
            rows16 = e0 + lax.iota(jnp.int32, 16)
            acc = jnp.zeros((16,), jnp.float32)
            for f in range(OUT):
                col = jnp.full((16,), f, jnp.int32)
                acc = acc + (plsc.load_gather(ra, [rows16, col])
                             * plsc.load_gather(rb, [rows16, col]))
            pv[pl.ds(e0, 16)] = acc
            return cc

        lax.fori_loop(0, (GRP * CH) // 16, dot16, 0)
        pltpu.sync_copy(pv, out.at[pl.ds(base * CH, GRP * CH)])
        return carry

    lax.fori_loop(0, TROWS // GRP, grp, 0)


_dot_call = pl.kernel(
    _dot_body,
    out_type=jax.ShapeDtypeStruct((EPAD,), jnp.float32),
    mesh=_mesh,
    compiler_params=pltpu.CompilerParams(use_tc_tiling_on_sc=False, needs_layout_passes=False),
    scratch_types=[
        pltpu.VMEM((GRP, CH), jnp.int32),
        pltpu.VMEM((GRP, CH), jnp.int32),
        pltpu.VMEM((GRP * CH, OUT), jnp.float32),
        pltpu.VMEM((GRP * CH, OUT), jnp.float32),
        pltpu.VMEM((GRP * CH,), jnp.float32),
        pltpu.SemaphoreType.DMA,
    ],
)


BN = 2000


def _prep_body(emb, W1, b1, W2, b2, d0, d1, z0_o, c_o, d2x_o, dfx_o, ch_o):
    x1 = jnp.maximum(
        jnp.dot(emb[...], W1[...], preferred_element_type=jnp.float32)
        + b1[...], 0.0)
    x0 = jnp.dot(x1, W2[...], preferred_element_type=jnp.float32) + b2[...]
    deg = d0[...] + d1[...] + 1.0
    dis = lax.rsqrt(deg)
    z0_o[...] = x0 * dis
    c_o[...] = ALPHA * dis * x0
    d2x_o[...] = jnp.broadcast_to((1.0 - ALPHA) * dis * dis, (BN, OUT))
    dfx_o[...] = jnp.broadcast_to((1.0 - ALPHA) * dis, (BN, OUT))
    ch_o[...] = ALPHA * x0


_prep_call = pl.pallas_call(
    _prep_body,
    grid=(N // BN,),
    in_specs=[
        pl.BlockSpec((BN, EMB), lambda i: (i, 0)),
        pl.BlockSpec((EMB, HID), lambda i: (0, 0)),
        pl.BlockSpec((1, HID), lambda i: (0, 0)),
        pl.BlockSpec((HID, OUT), lambda i: (0, 0)),
        pl.BlockSpec((1, OUT), lambda i: (0, 0)),
        pl.BlockSpec((BN, 1), lambda i: (i, 0)),
        pl.BlockSpec((BN, 1), lambda i: (i, 0)),
    ],
    out_specs=[pl.BlockSpec((BN, OUT), lambda i: (i, 0))] * 5,
    out_shape=[jax.ShapeDtypeStruct((N, OUT), jnp.float32)] * 5,
)


def _blend_body(s0, s1, z, coef, addt, o):
    o[...] = coef[...] * (s0[...] + s1[...] + z[...]) + addt[...]


_blend_call = pl.pallas_call(
    _blend_body,
    grid=(N // BN,),
    in_specs=[pl.BlockSpec((BN, OUT), lambda i: (i, 0))] * 5,
    out_specs=pl.BlockSpec((BN, OUT), lambda i: (i, 0)),
    out_shape=jax.ShapeDtypeStruct((N, OUT), jnp.float32),
)


def kernel(edge_index, emb, W1, b1, W2, b2):
    src = edge_index[0]
    dst = edge_index[1]
    pad = EPAD - E
    srcp = jnp.concatenate(
        [src, jnp.zeros((pad,), jnp.int32)]).reshape(EROWS, CH)
    dstp = jnp.concatenate(
        [dst, jnp.full((pad,), N, jnp.int32)]).reshape(EROWS, CH)
    dstp0 = jnp.concatenate(
        [dst, jnp.zeros((pad,), jnp.int32)]).reshape(EROWS, CH)

    zeros1 = jnp.zeros((DZBLK,), jnp.float32)
    zeros2 = jnp.zeros((ZBLK, OUT), jnp.float32)

    dparts = _deg_call(dstp, zeros1)
    d0 = dparts[0].reshape(DEG_R, 1)
    d1 = dparts[1].reshape(DEG_R, 1)

    z0, c, d2x, dfx, ch = _prep_call(
        emb, W1, b1.reshape(1, HID), W2, b2.reshape(1, OUT), d0, d1)

    z = z0
    for k in range(K):
        S = _scat_call(srcp, dstp, z, zeros2)
        if k < K - 1:
            z = _blend_call(S[0], S[1], z, d2x, c)
        else:
            x = _blend_call(S[0], S[1], z, dfx, ch)

    predp = _dot_call(srcp, dstp0, x)
    return predp[:E]


# speedup vs baseline: 29.2981x; 1.1576x over previous
import jax
import jax.numpy as jnp
from jax import lax
from jax.experimental import pallas as pl
from jax.experimental.pallas import tpu as pltpu
from jax.experimental.pallas import tpu_sc as plsc

N = 100000
E = 3200000
EMB = 32
HID = 8
OUT = 16
K = 10
ALPHA = 0.1

NC = 2
NS = 16
NW = NC * NS

CH = 128
GRP = 4
EPAD = 3211264
DGRP = 4
EROWS = EPAD // CH
TROWS = EROWS // NW

ACC_R = 100096
ZBLK = ACC_R // NS

DEG_R = 100352
DZBLK = DEG_R // NS

_mesh = plsc.VectorSubcoreMesh(core_axis_name="c", subcore_axis_name="s")

def _deg_body(dstp, zeros1, out, dacc, idx_v, ones_v):
    c = lax.axis_index("c")
    s = lax.axis_index("s")
    w = c * NS + s
    for i in range(CH // 16):
        ones_v[pl.ds(i * 16, 16)] = jnp.full((16,), 1.0, jnp.float32)
    pltpu.sync_copy(zeros1, dacc.at[pl.ds(s * DZBLK, DZBLK)])
    plsc.subcore_barrier()
    row0 = w * TROWS

    def grp(g, carry):
        pltpu.sync_copy(dstp.at[pl.ds(row0 + g * 16, 16)], idx_v)
        for j in range(16):
            pltpu.sync_copy(ones_v, dacc.at[idx_v.at[j]], add=True)
        return carry

    lax.fori_loop(0, TROWS // 16, grp, 0)
    plsc.subcore_barrier()
    pltpu.sync_copy(dacc.at[pl.ds(s * DZBLK, DZBLK)],
                    out.at[c].at[pl.ds(s * DZBLK, DZBLK)])


_deg_call = pl.kernel(
    _deg_body,
    out_type=jax.ShapeDtypeStruct((NC, DEG_R), jnp.float32),
    mesh=_mesh,
    compiler_params=pltpu.CompilerParams(use_tc_tiling_on_sc=False, needs_layout_passes=False),
    scratch_types=[
        pltpu.VMEM_SHARED((DEG_R,), jnp.float32),
        pltpu.VMEM((16, CH), jnp.int32),
        pltpu.VMEM((CH,), jnp.float32),
    ],
)


def _scat_body(srcp, dstp, z, zeros2, out, acc,
               si0, di0, rows0, si1, di1, rows1, sem0, sem1):
    c = lax.axis_index("c")
    s = lax.axis_index("s")
    w = c * NS + s
    pltpu.sync_copy(zeros2, acc.at[pl.ds(s * ZBLK, ZBLK)])
    plsc.subcore_barrier()
    row0 = w * TROWS
    NG = TROWS // GRP

    def start(g, sib, dib, rowsb, sem):
        base = row0 + g * GRP
        pltpu.sync_copy(srcp.at[pl.ds(base, GRP)], sib)
        pltpu.sync_copy(dstp.at[pl.ds(base, GRP)], dib)
        for j in range(GRP):
            pltpu.async_copy(z.at[sib.at[j]], rowsb.at[pl.ds(j * CH, CH)],
                             sem)

    def finish(rowsb, sem):
        pltpu.make_async_copy(z.at[pl.ds(0, GRP * CH)], rowsb, sem).wait()

    def scat(dib, rowsb):
        for j in range(GRP):
            pltpu.sync_copy(rowsb.at[pl.ds(j * CH, CH)], acc.at[dib.at[j]],
                            add=True)

    start(0, si0, di0, rows0, sem0)

    def body(t2, carry):
        g1 = t2 * 2 + 1
        start(g1, si1, di1, rows1, sem1)
        finish(rows0, sem0)
        scat(di0, rows0)

        @pl.when(g1 + 1 < NG)
        def _():
            start(g1 + 1, si0, di0, rows0, sem0)

        finish(rows1, sem1)
        scat(di1, rows1)
        return carry

    lax.fori_loop(0, NG // 2, body, 0)
    plsc.subcore_barrier()
    pltpu.sync_copy(acc.at[pl.ds(s * ZBLK, ZBLK)],
                    out.at[c].at[pl.ds(s * ZBLK, ZBLK)])


_scat_call = pl.kernel(
    _scat_body,
    out_type=jax.ShapeDtypeStruct((NC, ACC_R, OUT), jnp.float32),
    mesh=_mesh,
    compiler_params=pltpu.CompilerParams(use_tc_tiling_on_sc=False, needs_layout_passes=False),
    scratch_types=[
        pltpu.VMEM_SHARED((ACC_R, OUT), jnp.float32),
        pltpu.VMEM((GRP, CH), jnp.int32),
        pltpu.VMEM((GRP, CH), jnp.int32),
        pltpu.VMEM((GRP * CH, OUT), jnp.float32),
        pltpu.VMEM((GRP, CH), jnp.int32),
        pltpu.VMEM((GRP, CH), jnp.int32),
        pltpu.VMEM((GRP * CH, OUT), jnp.float32),
        pltpu.SemaphoreType.DMA,
        pltpu.SemaphoreType.DMA,
    ],
)


def _dot_body(srcp, dstp, x, out,
              si0, di0, ra0, rb0, si1, di1, ra1, rb1, pv, sem0, sem1):
    c = lax.axis_index("c")
    s = lax.axis_index("s")
    w = c * NS + s
    row0 = w * TROWS
    NG = TROWS // DGRP

    def start(g, sib, dib, rab, rbb, sem):
        base = row0 + g * DGRP
        pltpu.sync_copy(srcp.at[pl.ds(base, DGRP)], sib)
        pltpu.sync_copy(dstp.at[pl.ds(base, DGRP)], dib)
        for j in range(DGRP):
            pltpu.async_copy(x.at[sib.at[j]], rab.at[pl.ds(j * CH, CH)], sem)
        for j in range(DGRP):
            pltpu.async_copy(x.at[dib.at[j]], rbb.at[pl.ds(j * CH, CH)], sem)

    def finish(rab, rbb, sem):
        pltpu.make_async_copy(x.at[pl.ds(0, DGRP * CH)], rab, sem).wait()
        pltpu.make_async_copy(x.at[pl.ds(0, DGRP * CH)], rbb, sem).wait()

    def compute(g, rab, rbb):
        def dot16(i, cc):
            e0 = i * 16
            rows16 = e0 + lax.iota(jnp.int32, 16)
            acc = jnp.zeros((16,), jnp.float32)
            for f in range(OUT):
                col = jnp.full((16,), f, jnp.int32)
                acc = acc + (plsc.load_gather(rab, [rows16, col])
                             * plsc.load_gather(rbb, [rows16, col]))
            pv[pl.ds(e0, 16)] = acc
            return cc

        lax.fori_loop(0, (DGRP * CH) // 16, dot16, 0)
        pltpu.sync_copy(pv, out.at[pl.ds((row0 + g * DGRP) * CH, DGRP * CH)])

    start(0, si0, di0, ra0, rb0, sem0)

    def body(t2, carry):
        g1 = t2 * 2 + 1
        start(g1, si1, di1, ra1, rb1, sem1)
        finish(ra0, rb0, sem0)
        compute(g1 - 1, ra0, rb0)

        @pl.when(g1 + 1 < NG)
        def _():
            start(g1 + 1, si0, di0, ra0, rb0, sem0)

        finish(ra1, rb1, sem1)
        compute(g1, ra1, rb1)
        return carry

    lax.fori_loop(0, NG // 2, body, 0)


_dot_call = pl.kernel(
    _dot_body,
    out_type=jax.ShapeDtypeStruct((EPAD,), jnp.float32),
    mesh=_mesh,
    compiler_params=pltpu.CompilerParams(use_tc_tiling_on_sc=False, needs_layout_passes=False),
    scratch_types=[
        pltpu.VMEM((DGRP, CH), jnp.int32),
        pltpu.VMEM((DGRP, CH), jnp.int32),
        pltpu.VMEM((DGRP * CH, OUT), jnp.float32),
        pltpu.VMEM((DGRP * CH, OUT), jnp.float32),
        pltpu.VMEM((DGRP, CH), jnp.int32),
        pltpu.VMEM((DGRP, CH), jnp.int32),
        pltpu.VMEM((DGRP * CH, OUT), jnp.float32),
        pltpu.VMEM((DGRP * CH, OUT), jnp.float32),
        pltpu.VMEM((DGRP * CH,), jnp.float32),
        pltpu.SemaphoreType.DMA,
        pltpu.SemaphoreType.DMA,
    ],
)


BN = 2000


def _prep_body(emb, W1, b1, W2, b2, d0, d1, z0_o, c_o, d2x_o, dfx_o, ch_o):
    x1 = jnp.maximum(
        jnp.dot(emb[...], W1[...], preferred_element_type=jnp.float32)
        + b1[...], 0.0)
    x0 = jnp.dot(x1, W2[...], preferred_element_type=jnp.float32) + b2[...]
    deg = d0[...] + d1[...] + 1.0
    dis = lax.rsqrt(deg)
    z0_o[...] = x0 * dis
    c_o[...] = ALPHA * dis * x0
    d2x_o[...] = jnp.broadcast_to((1.0 - ALPHA) * dis * dis, (BN, OUT))
    dfx_o[...] = jnp.broadcast_to((1.0 - ALPHA) * dis, (BN, OUT))
    ch_o[...] = ALPHA * x0


_prep_call = pl.pallas_call(
    _prep_body,
    grid=(N // BN,),
    in_specs=[
        pl.BlockSpec((BN, EMB), lambda i: (i, 0)),
        pl.BlockSpec((EMB, HID), lambda i: (0, 0)),
        pl.BlockSpec((1, HID), lambda i: (0, 0)),
        pl.BlockSpec((HID, OUT), lambda i: (0, 0)),
        pl.BlockSpec((1, OUT), lambda i: (0, 0)),
        pl.BlockSpec((BN, 1), lambda i: (i, 0)),
        pl.BlockSpec((BN, 1), lambda i: (i, 0)),
    ],
    out_specs=[pl.BlockSpec((BN, OUT), lambda i: (i, 0))] * 5,
    out_shape=[jax.ShapeDtypeStruct((N, OUT), jnp.float32)] * 5,
)


def _blend_body(s0, s1, z, coef, addt, o):
    o[...] = coef[...] * (s0[...] + s1[...] + z[...]) + addt[...]


_blend_call = pl.pallas_call(
    _blend_body,
    grid=(N // BN,),
    in_specs=[pl.BlockSpec((BN, OUT), lambda i: (i, 0))] * 5,
    out_specs=pl.BlockSpec((BN, OUT), lambda i: (i, 0)),
    out_shape=jax.ShapeDtypeStruct((N, OUT), jnp.float32),
)


def kernel(edge_index, emb, W1, b1, W2, b2):
    src = edge_index[0]
    dst = edge_index[1]
    pad = EPAD - E
    srcp = jnp.concatenate(
        [src, jnp.zeros((pad,), jnp.int32)]).reshape(EROWS, CH)
    dstp = jnp.concatenate(
        [dst, jnp.full((pad,), N, jnp.int32)]).reshape(EROWS, CH)
    dstp0 = jnp.concatenate(
        [dst, jnp.zeros((pad,), jnp.int32)]).reshape(EROWS, CH)

    zeros1 = jnp.zeros((DZBLK,), jnp.float32)
    zeros2 = jnp.zeros((ZBLK, OUT), jnp.float32)

    dparts = _deg_call(dstp, zeros1)
    d0 = dparts[0].reshape(DEG_R, 1)
    d1 = dparts[1].reshape(DEG_R, 1)

    z0, c, d2x, dfx, ch = _prep_call(
        emb, W1, b1.reshape(1, HID), W2, b2.reshape(1, OUT), d0, d1)

    z = z0
    for k in range(K):
        S = _scat_call(srcp, dstp, z, zeros2)
        if k < K - 1:
            z = _blend_call(S[0], S[1], z, d2x, c)
        else:
            x = _blend_call(S[0], S[1], z, dfx, ch)

    predp = _dot_call(srcp, dstp0, x)
    return predp[:E]


# speedup vs baseline: 31.2849x; 1.0678x over previous
import jax
import jax.numpy as jnp
from jax import lax
from jax.experimental import pallas as pl
from jax.experimental.pallas import tpu as pltpu
from jax.experimental.pallas import tpu_sc as plsc

N = 100000
E = 3200000
EMB = 32
HID = 8
OUT = 16
K = 10
ALPHA = 0.1

NC = 2
NS = 16
NW = NC * NS

CH = 128
GRP = 4
EPAD = 3211264
DGRP = 4
EROWS = EPAD // CH
TROWS = EROWS // NW

ACC_R = 100096
ZBLK = ACC_R // NS

DEG_R = 100352
DZBLK = DEG_R // NS

_mesh = plsc.VectorSubcoreMesh(core_axis_name="c", subcore_axis_name="s")

def _deg_body(dstp, zeros1, out, dacc, idx_v, ones_v):
    c = lax.axis_index("c")
    s = lax.axis_index("s")
    w = c * NS + s
    for i in range(CH // 16):
        ones_v[pl.ds(i * 16, 16)] = jnp.full((16,), 1.0, jnp.float32)
    pltpu.sync_copy(zeros1, dacc.at[pl.ds(s * DZBLK, DZBLK)])
    plsc.subcore_barrier()
    row0 = w * TROWS

    def grp(g, carry):
        pltpu.sync_copy(dstp.at[pl.ds(row0 + g * 16, 16)], idx_v)
        for j in range(16):
            pltpu.sync_copy(ones_v, dacc.at[idx_v.at[j]], add=True)
        return carry

    lax.fori_loop(0, TROWS // 16, grp, 0)
    plsc.subcore_barrier()
    pltpu.sync_copy(dacc.at[pl.ds(s * DZBLK, DZBLK)],
                    out.at[c].at[pl.ds(s * DZBLK, DZBLK)])


_deg_call = pl.kernel(
    _deg_body,
    out_type=jax.ShapeDtypeStruct((NC, DEG_R), jnp.float32),
    mesh=_mesh,
    compiler_params=pltpu.CompilerParams(use_tc_tiling_on_sc=False, needs_layout_passes=False),
    scratch_types=[
        pltpu.VMEM_SHARED((DEG_R,), jnp.float32),
        pltpu.VMEM((16, CH), jnp.int32),
        pltpu.VMEM((CH,), jnp.float32),
    ],
)


def _scat_body(srcp, dstp, z, zeros2, out, acc,
               si0, di0, rows0, si1, di1, rows1,
               sem0, sem1, semi0, semi1):
    c = lax.axis_index("c")
    s = lax.axis_index("s")
    w = c * NS + s
    pltpu.sync_copy(zeros2, acc.at[pl.ds(s * ZBLK, ZBLK)])
    plsc.subcore_barrier()
    row0 = w * TROWS
    NG = TROWS // GRP

    def idx_start(g, sib, dib, semi):
        base = row0 + g * GRP
        pltpu.async_copy(srcp.at[pl.ds(base, GRP)], sib, semi)
        pltpu.async_copy(dstp.at[pl.ds(base, GRP)], dib, semi)

    def idx_wait(sib, dib, semi):
        pltpu.make_async_copy(srcp.at[pl.ds(0, GRP)], sib, semi).wait()
        pltpu.make_async_copy(dstp.at[pl.ds(0, GRP)], dib, semi).wait()

    def gath(sib, rowsb, sem):
        for j in range(GRP):
            pltpu.async_copy(z.at[sib.at[j]], rowsb.at[pl.ds(j * CH, CH)],
                             sem)

    def finish(rowsb, sem):
        pltpu.make_async_copy(z.at[pl.ds(0, GRP * CH)], rowsb, sem).wait()

    def scat(dib, rowsb):
        for j in range(GRP):
            pltpu.sync_copy(rowsb.at[pl.ds(j * CH, CH)], acc.at[dib.at[j]],
                            add=True)

    idx_start(0, si0, di0, semi0)
    idx_start(1, si1, di1, semi1)
    idx_wait(si0, di0, semi0)
    gath(si0, rows0, sem0)

    def body(t2, carry):
        g1 = t2 * 2 + 1
        idx_wait(si1, di1, semi1)
        gath(si1, rows1, sem1)
        finish(rows0, sem0)
        scat(di0, rows0)

        @pl.when(g1 + 1 < NG)
        def _():
            idx_start(g1 + 1, si0, di0, semi0)

        finish(rows1, sem1)
        scat(di1, rows1)

        @pl.when(g1 + 2 < NG)
        def _():
            idx_start(g1 + 2, si1, di1, semi1)

        @pl.when(g1 + 1 < NG)
        def _():
            idx_wait(si0, di0, semi0)
            gath(si0, rows0, sem0)

        return carry

    lax.fori_loop(0, NG // 2, body, 0)
    plsc.subcore_barrier()
    pltpu.sync_copy(acc.at[pl.ds(s * ZBLK, ZBLK)],
                    out.at[c].at[pl.ds(s * ZBLK, ZBLK)])


_scat_call = pl.kernel(
    _scat_body,
    out_type=jax.ShapeDtypeStruct((NC, ACC_R, OUT), jnp.float32),
    mesh=_mesh,
    compiler_params=pltpu.CompilerParams(use_tc_tiling_on_sc=False, needs_layout_passes=False),
    scratch_types=[
        pltpu.VMEM_SHARED((ACC_R, OUT), jnp.float32),
        pltpu.VMEM((GRP, CH), jnp.int32),
        pltpu.VMEM((GRP, CH), jnp.int32),
        pltpu.VMEM((GRP * CH, OUT), jnp.float32),
        pltpu.VMEM((GRP, CH), jnp.int32),
        pltpu.VMEM((GRP, CH), jnp.int32),
        pltpu.VMEM((GRP * CH, OUT), jnp.float32),
        pltpu.SemaphoreType.DMA,
        pltpu.SemaphoreType.DMA,
        pltpu.SemaphoreType.DMA,
        pltpu.SemaphoreType.DMA,
    ],
)


def _dot_body(srcp, dstp, x, out,
              si0, di0, ra0, rb0, si1, di1, ra1, rb1, pv,
              sem0, sem1, semi0, semi1):
    c = lax.axis_index("c")
    s = lax.axis_index("s")
    w = c * NS + s
    row0 = w * TROWS
    NG = TROWS // DGRP

    def idx_start(g, sib, dib, semi):
        base = row0 + g * DGRP
        pltpu.async_copy(srcp.at[pl.ds(base, DGRP)], sib, semi)
        pltpu.async_copy(dstp.at[pl.ds(base, DGRP)], dib, semi)

    def idx_wait(sib, dib, semi):
        pltpu.make_async_copy(srcp.at[pl.ds(0, DGRP)], sib, semi).wait()
        pltpu.make_async_copy(dstp.at[pl.ds(0, DGRP)], dib, semi).wait()

    def gath(sib, dib, rab, rbb, sem):
        for j in range(DGRP):
            pltpu.async_copy(x.at[sib.at[j]], rab.at[pl.ds(j * CH, CH)], sem)
        for j in range(DGRP):
            pltpu.async_copy(x.at[dib.at[j]], rbb.at[pl.ds(j * CH, CH)], sem)

    def finish(rab, rbb, sem):
        pltpu.make_async_copy(x.at[pl.ds(0, DGRP * CH)], rab, sem).wait()
        pltpu.make_async_copy(x.at[pl.ds(0, DGRP * CH)], rbb, sem).wait()

    def compute(g, rab, rbb):
        def dot16(i, cc):
            e0 = i * 16
            rows16 = e0 + lax.iota(jnp.int32, 16)
            acc = jnp.zeros((16,), jnp.float32)
            for f in range(OUT):
                col = jnp.full((16,), f, jnp.int32)
                acc = acc + (plsc.load_gather(rab, [rows16, col])
                             * plsc.load_gather(rbb, [rows16, col]))
            pv[pl.ds(e0, 16)] = acc
            return cc

        lax.fori_loop(0, (DGRP * CH) // 16, dot16, 0)
        pltpu.sync_copy(pv, out.at[pl.ds((row0 + g * DGRP) * CH, DGRP * CH)])

    idx_start(0, si0, di0, semi0)
    idx_start(1, si1, di1, semi1)
    idx_wait(si0, di0, semi0)
    gath(si0, di0, ra0, rb0, sem0)

    def body(t2, carry):
        g1 = t2 * 2 + 1
        idx_wait(si1, di1, semi1)
        gath(si1, di1, ra1, rb1, sem1)
        finish(ra0, rb0, sem0)
        compute(g1 - 1, ra0, rb0)

        @pl.when(g1 + 1 < NG)
        def _():
            idx_start(g1 + 1, si0, di0, semi0)

        finish(ra1, rb1, sem1)
        compute(g1, ra1, rb1)

        @pl.when(g1 + 2 < NG)
        def _():
            idx_start(g1 + 2, si1, di1, semi1)

        @pl.when(g1 + 1 < NG)
        def _():
            idx_wait(si0, di0, semi0)
            gath(si0, di0, ra0, rb0, sem0)

        return carry

    lax.fori_loop(0, NG // 2, body, 0)


_dot_call = pl.kernel(
    _dot_body,
    out_type=jax.ShapeDtypeStruct((EPAD,), jnp.float32),
    mesh=_mesh,
    compiler_params=pltpu.CompilerParams(use_tc_tiling_on_sc=False, needs_layout_passes=False),
    scratch_types=[
        pltpu.VMEM((DGRP, CH), jnp.int32),
        pltpu.VMEM((DGRP, CH), jnp.int32),
        pltpu.VMEM((DGRP * CH, OUT), jnp.float32),
        pltpu.VMEM((DGRP * CH, OUT), jnp.float32),
        pltpu.VMEM((DGRP, CH), jnp.int32),
        pltpu.VMEM((DGRP, CH), jnp.int32),
        pltpu.VMEM((DGRP * CH, OUT), jnp.float32),
        pltpu.VMEM((DGRP * CH, OUT), jnp.float32),
        pltpu.VMEM((DGRP * CH,), jnp.float32),
        pltpu.SemaphoreType.DMA,
        pltpu.SemaphoreType.DMA,
        pltpu.SemaphoreType.DMA,
        pltpu.SemaphoreType.DMA,
    ],
)


BN = 2000


def _prep_body(emb, W1, b1, W2, b2, d0, d1, z0_o, c_o, d2x_o, dfx_o, ch_o):
    x1 = jnp.maximum(
        jnp.dot(emb[...], W1[...], preferred_element_type=jnp.float32)
        + b1[...], 0.0)
    x0 = jnp.dot(x1, W2[...], preferred_element_type=jnp.float32) + b2[...]
    deg = d0[...] + d1[...] + 1.0
    dis = lax.rsqrt(deg)
    z0_o[...] = x0 * dis
    c_o[...] = ALPHA * dis * x0
    d2x_o[...] = jnp.broadcast_to((1.0 - ALPHA) * dis * dis, (BN, OUT))
    dfx_o[...] = jnp.broadcast_to((1.0 - ALPHA) * dis, (BN, OUT))
    ch_o[...] = ALPHA * x0


_prep_call = pl.pallas_call(
    _prep_body,
    grid=(N // BN,),
    in_specs=[
        pl.BlockSpec((BN, EMB), lambda i: (i, 0)),
        pl.BlockSpec((EMB, HID), lambda i: (0, 0)),
        pl.BlockSpec((1, HID), lambda i: (0, 0)),
        pl.BlockSpec((HID, OUT), lambda i: (0, 0)),
        pl.BlockSpec((1, OUT), lambda i: (0, 0)),
        pl.BlockSpec((BN, 1), lambda i: (i, 0)),
        pl.BlockSpec((BN, 1), lambda i: (i, 0)),
    ],
    out_specs=[pl.BlockSpec((BN, OUT), lambda i: (i, 0))] * 5,
    out_shape=[jax.ShapeDtypeStruct((N, OUT), jnp.float32)] * 5,
)


def _blend_body(s0, s1, z, coef, addt, o):
    o[...] = coef[...] * (s0[...] + s1[...] + z[...]) + addt[...]


_blend_call = pl.pallas_call(
    _blend_body,
    grid=(N // BN,),
    in_specs=[pl.BlockSpec((BN, OUT), lambda i: (i, 0))] * 5,
    out_specs=pl.BlockSpec((BN, OUT), lambda i: (i, 0)),
    out_shape=jax.ShapeDtypeStruct((N, OUT), jnp.float32),
)


def kernel(edge_index, emb, W1, b1, W2, b2):
    src = edge_index[0]
    dst = edge_index[1]
    pad = EPAD - E
    srcp = jnp.concatenate(
        [src, jnp.zeros((pad,), jnp.int32)]).reshape(EROWS, CH)
    dstp = jnp.concatenate(
        [dst, jnp.full((pad,), N, jnp.int32)]).reshape(EROWS, CH)
    dstp0 = jnp.concatenate(
        [dst, jnp.zeros((pad,), jnp.int32)]).reshape(EROWS, CH)

    zeros1 = jnp.zeros((DZBLK,), jnp.float32)
    zeros2 = jnp.zeros((ZBLK, OUT), jnp.float32)

    dparts = _deg_call(dstp, zeros1)
    d0 = dparts[0].reshape(DEG_R, 1)
    d1 = dparts[1].reshape(DEG_R, 1)

    z0, c, d2x, dfx, ch = _prep_call(
        emb, W1, b1.reshape(1, HID), W2, b2.reshape(1, OUT), d0, d1)

    z = z0
    for k in range(K):
        S = _scat_call(srcp, dstp, z, zeros2)
        if k < K - 1:
            z = _blend_call(S[0], S[1], z, d2x, c)
        else:
            x = _blend_call(S[0], S[1], z, dfx, ch)

    predp = _dot_call(srcp, dstp0, x)
    return predp[:E]


# speedup vs baseline: 41.9997x; 1.3425x over previous
import jax
import jax.numpy as jnp
from jax import lax
from jax.experimental import pallas as pl
from jax.experimental.pallas import tpu as pltpu
from jax.experimental.pallas import tpu_sc as plsc

N = 100000
E = 3200000
EMB = 32
HID = 8
OUT = 16
K = 10
ALPHA = 0.1

NC = 2
NS = 16
NW = NC * NS

CH = 128
GRP = 4
DGRP = 8
EPAD = 3211264
EROWS = EPAD // CH
TROWS = EROWS // NW

ACC_R = 100352
ZBLK = ACC_R // NS
CB = 64
NCB = ZBLK // CB

DEG_R = 100352
DZBLK = DEG_R // NS

_mesh = plsc.VectorSubcoreMesh(core_axis_name="c", subcore_axis_name="s")
_params = pltpu.CompilerParams(use_tc_tiling_on_sc=False,
                               needs_layout_passes=False)


def _deg_body(dstp, zeros1, out, dacc, idx_v, ones_v):
    c = lax.axis_index("c")
    s = lax.axis_index("s")
    w = c * NS + s
    for i in range(CH // 16):
        ones_v[pl.ds(i * 16, 16)] = jnp.full((16,), 1.0, jnp.float32)
    pltpu.sync_copy(zeros1, dacc.at[pl.ds(s * DZBLK, DZBLK)])
    plsc.subcore_barrier()
    row0 = w * TROWS

    def grp(g, carry):
        pltpu.sync_copy(dstp.at[pl.ds(row0 + g * 16, 16)], idx_v)
        for j in range(16):
            pltpu.sync_copy(ones_v, dacc.at[idx_v.at[j]], add=True)
        return carry

    lax.fori_loop(0, TROWS // 16, grp, 0)
    plsc.subcore_barrier()
    pltpu.sync_copy(dacc.at[pl.ds(s * DZBLK, DZBLK)],
                    out.at[c].at[pl.ds(s * DZBLK, DZBLK)])


_deg_call = pl.kernel(
    _deg_body,
    out_type=jax.ShapeDtypeStruct((NC, DEG_R), jnp.float32),
    mesh=_mesh,
    compiler_params=_params,
    scratch_types=[
        pltpu.VMEM_SHARED((DEG_R,), jnp.float32),
        pltpu.VMEM((16, CH), jnp.int32),
        pltpu.VMEM((CH,), jnp.float32),
    ],
)


def _scatter_phase(srcp, dstp, ztab, acc, si0, di0, rows0, si1, di1, rows1,
                   sem0, sem1, semi0, semi1, row0):
    NG = TROWS // GRP

    def idx_start(g, sib, dib, semi):
        base = row0 + g * GRP
        pltpu.async_copy(srcp.at[pl.ds(base, GRP)], sib, semi)
        pltpu.async_copy(dstp.at[pl.ds(base, GRP)], dib, semi)

    def idx_wait(sib, dib, semi):
        pltpu.make_async_copy(srcp.at[pl.ds(0, GRP)], sib, semi).wait()
        pltpu.make_async_copy(dstp.at[pl.ds(0, GRP)], dib, semi).wait()

    def gath(sib, rowsb, sem):
        for j in range(GRP):
            pltpu.async_copy(ztab.at[sib.at[j]], rowsb.at[pl.ds(j * CH, CH)],
                             sem)

    def finish(rowsb, sem):
        pltpu.make_async_copy(ztab.at[pl.ds(0, GRP * CH)], rowsb, sem).wait()

    def scat(dib, rowsb):
        for j in range(GRP):
            pltpu.sync_copy(rowsb.at[pl.ds(j * CH, CH)], acc.at[dib.at[j]],
                            add=True)

    idx_start(0, si0, di0, semi0)
    idx_start(1, si1, di1, semi1)
    idx_wait(si0, di0, semi0)
    gath(si0, rows0, sem0)

    def body(t2, carry):
        g1 = t2 * 2 + 1
        idx_wait(si1, di1, semi1)
        gath(si1, rows1, sem1)
        finish(rows0, sem0)
        scat(di0, rows0)

        @pl.when(g1 + 1 < NG)
        def _():
            idx_start(g1 + 1, si0, di0, semi0)

        finish(rows1, sem1)
        scat(di1, rows1)

        @pl.when(g1 + 2 < NG)
        def _():
            idx_start(g1 + 2, si1, di1, semi1)

        @pl.when(g1 + 1 < NG)
        def _():
            idx_wait(si0, di0, semi0)
            gath(si0, rows0, sem0)

        return carry

    lax.fori_loop(0, NG // 2, body, 0)


def _blend_phase(s0r, s1r, zpr, coef, addt, zout, zb0, A, B, semb0, semb1):
    def bl_start(i, bufs, semb):
        r0 = zb0 + i * CB
        pltpu.async_copy(s0r.at[pl.ds(r0, CB)], bufs[0], semb)
        pltpu.async_copy(s1r.at[pl.ds(r0, CB)], bufs[1], semb)
        pltpu.async_copy(zpr.at[pl.ds(r0, CB)], bufs[2], semb)
        pltpu.async_copy(coef.at[pl.ds(r0, CB)], bufs[3], semb)
        pltpu.async_copy(addt.at[pl.ds(r0, CB)], bufs[4], semb)

    def bl_fin(i, bufs, semb):
        for bref in bufs:
            pltpu.make_async_copy(coef.at[pl.ds(0, CB)], bref, semb).wait()

        def rowf(r, cc):
            bufs[2][r] = (bufs[3][r] * (bufs[0][r] + bufs[1][r] + bufs[2][r])
                          + bufs[4][r])
            return cc

        lax.fori_loop(0, CB, rowf, 0)
        pltpu.sync_copy(bufs[2], zout.at[pl.ds(zb0 + i * CB, CB)])

    bl_start(0, A, semb0)

    def blbody(t2, cc):
        i1 = t2 * 2 + 1
        bl_start(i1, B, semb1)
        bl_fin(i1 - 1, A, semb0)

        @pl.when(i1 + 1 < NCB)
        def _():
            bl_start(i1 + 1, A, semb0)

        bl_fin(i1, B, semb1)
        return cc

    lax.fori_loop(0, NCB // 2, blbody, 0)


def _scat_body(srcp, dstp, z, zeros2, out, acc,
               si0, di0, rows0, si1, di1, rows1,
               sem0, sem1, semi0, semi1):
    c = lax.axis_index("c")
    s = lax.axis_index("s")
    w = c * NS + s
    pltpu.sync_copy(zeros2, acc.at[pl.ds(s * ZBLK, ZBLK)])
    plsc.subcore_barrier()
    _scatter_phase(srcp, dstp, z, acc, si0, di0, rows0, si1, di1, rows1,
                   sem0, sem1, semi0, semi1, w * TROWS)
    plsc.subcore_barrier()
    pltpu.sync_copy(acc.at[pl.ds(s * ZBLK, ZBLK)],
                    out.at[c].at[pl.ds(s * ZBLK, ZBLK)])


_scat_call = pl.kernel(
    _scat_body,
    out_type=jax.ShapeDtypeStruct((NC, ACC_R, OUT), jnp.float32),
    mesh=_mesh,
    compiler_params=_params,
    scratch_types=[
        pltpu.VMEM_SHARED((ACC_R, OUT), jnp.float32),
        pltpu.VMEM((GRP, CH), jnp.int32),
        pltpu.VMEM((GRP, CH), jnp.int32),
        pltpu.VMEM((GRP * CH, OUT), jnp.float32),
        pltpu.VMEM((GRP, CH), jnp.int32),
        pltpu.VMEM((GRP, CH), jnp.int32),
        pltpu.VMEM((GRP * CH, OUT), jnp.float32),
        pltpu.SemaphoreType.DMA,
        pltpu.SemaphoreType.DMA,
        pltpu.SemaphoreType.DMA,
        pltpu.SemaphoreType.DMA,
    ],
)


def _step_body(srcp, dstp, Sprev, zprev, coef, addt, zeros2,
               out_S, out_z, acc,
               si0, di0, rows0, si1, di1, rows1,
               a0, a1, a2, a3, a4, b0, b1, b2, b3, b4,
               sem0, sem1, semi0, semi1, semb0, semb1):
    c = lax.axis_index("c")
    s = lax.axis_index("s")
    w = c * NS + s
    zb0 = s * ZBLK
    pltpu.sync_copy(zeros2, acc.at[pl.ds(zb0, ZBLK)])
    _blend_phase(Sprev.at[0], Sprev.at[1], zprev.at[c], coef, addt,
                 out_z.at[c], zb0,
                 (a0, a1, a2, a3, a4), (b0, b1, b2, b3, b4), semb0, semb1)
    plsc.subcore_barrier()
    _scatter_phase(srcp, dstp, out_z.at[c], acc,
                   si0, di0, rows0, si1, di1, rows1,
                   sem0, sem1, semi0, semi1, w * TROWS)
    plsc.subcore_barrier()
    pltpu.sync_copy(acc.at[pl.ds(zb0, ZBLK)],
                    out_S.at[c].at[pl.ds(zb0, ZBLK)])


_step_call = pl.kernel(
    _step_body,
    out_type=(jax.ShapeDtypeStruct((NC, ACC_R, OUT), jnp.float32),
              jax.ShapeDtypeStruct((NC, ACC_R, OUT), jnp.float32)),
    mesh=_mesh,
    compiler_params=_params,
    scratch_types=[
        pltpu.VMEM_SHARED((ACC_R, OUT), jnp.float32),
        pltpu.VMEM((GRP, CH), jnp.int32),
        pltpu.VMEM((GRP, CH), jnp.int32),
        pltpu.VMEM((GRP * CH, OUT), jnp.float32),
        pltpu.VMEM((GRP, CH), jnp.int32),
        pltpu.VMEM((GRP, CH), jnp.int32),
        pltpu.VMEM((GRP * CH, OUT), jnp.float32),
        pltpu.VMEM((CB, OUT), jnp.float32),
        pltpu.VMEM((CB, OUT), jnp.float32),
        pltpu.VMEM((CB, OUT), jnp.float32),
        pltpu.VMEM((CB, OUT), jnp.float32),
        pltpu.VMEM((CB, OUT), jnp.float32),
        pltpu.VMEM((CB, OUT), jnp.float32),
        pltpu.VMEM((CB, OUT), jnp.float32),
        pltpu.VMEM((CB, OUT), jnp.float32),
        pltpu.VMEM((CB, OUT), jnp.float32),
        pltpu.VMEM((CB, OUT), jnp.float32),
        pltpu.SemaphoreType.DMA,
        pltpu.SemaphoreType.DMA,
        pltpu.SemaphoreType.DMA,
        pltpu.SemaphoreType.DMA,
        pltpu.SemaphoreType.DMA,
        pltpu.SemaphoreType.DMA,
    ],
)


def _dot_body(srcp, dstp, Sprev, zprev, coef, addt, out, out_x,
              si0, di0, ra0, rb0, si1, di1, ra1, rb1, pv,
              a0, a1, a2, a3, a4, b0, b1, b2, b3, b4,
              sem0, sem1, semi0, semi1, semb0, semb1):
    c = lax.axis_index("c")
    s = lax.axis_index("s")
    w = c * NS + s
    row0 = w * TROWS
    NG = TROWS // DGRP
    _blend_phase(Sprev.at[0], Sprev.at[1], zprev.at[c], coef, addt,
                 out_x.at[c], s * ZBLK,
                 (a0, a1, a2, a3, a4), (b0, b1, b2, b3, b4), semb0, semb1)
    plsc.subcore_barrier()
    x = out_x.at[c]

    def idx_start(g, sib, dib, semi):
        base = row0 + g * DGRP
        pltpu.async_copy(srcp.at[pl.ds(base, DGRP)], sib, semi)
        pltpu.async_copy(dstp.at[pl.ds(base, DGRP)], dib, semi)

    def idx_wait(sib, dib, semi):
        pltpu.make_async_copy(srcp.at[pl.ds(0, DGRP)], sib, semi).wait()
        pltpu.make_async_copy(dstp.at[pl.ds(0, DGRP)], dib, semi).wait()

    def gath(sib, dib, rab, rbb, sem):
        for j in range(DGRP):
            pltpu.async_copy(x.at[sib.at[j]], rab.at[pl.ds(j * CH, CH)], sem)
        for j in range(DGRP):
            pltpu.async_copy(x.at[dib.at[j]], rbb.at[pl.ds(j * CH, CH)], sem)

    def finish(rab, rbb, sem):
        pltpu.make_async_copy(x.at[pl.ds(0, DGRP * CH)], rab, sem).wait()
        pltpu.make_async_copy(x.at[pl.ds(0, DGRP * CH)], rbb, sem).wait()

    def compute(g, rab, rbb):
        def dot16(i, cc):
            e0 = i * 16
            rows16 = e0 + lax.iota(jnp.int32, 16)
            acc = jnp.zeros((16,), jnp.float32)
            for f in range(OUT):
                col = jnp.full((16,), f, jnp.int32)
                acc = acc + (plsc.load_gather(rab, [rows16, col])
                             * plsc.load_gather(rbb, [rows16, col]))
            pv[pl.ds(e0, 16)] = acc
            return cc

        lax.fori_loop(0, (DGRP * CH) // 16, dot16, 0)
        pltpu.sync_copy(pv, out.at[pl.ds((row0 + g * DGRP) * CH, DGRP * CH)])

    idx_start(0, si0, di0, semi0)
    idx_start(1, si1, di1, semi1)
    idx_wait(si0, di0, semi0)
    gath(si0, di0, ra0, rb0, sem0)

    def body(t2, carry):
        g1 = t2 * 2 + 1
        idx_wait(si1, di1, semi1)
        gath(si1, di1, ra1, rb1, sem1)
        finish(ra0, rb0, sem0)
        compute(g1 - 1, ra0, rb0)

        @pl.when(g1 + 1 < NG)
        def _():
            idx_start(g1 + 1, si0, di0, semi0)

        finish(ra1, rb1, sem1)
        compute(g1, ra1, rb1)

        @pl.when(g1 + 2 < NG)
        def _():
            idx_start(g1 + 2, si1, di1, semi1)

        @pl.when(g1 + 1 < NG)
        def _():
            idx_wait(si0, di0, semi0)
            gath(si0, di0, ra0, rb0, sem0)

        return carry

    lax.fori_loop(0, NG // 2, body, 0)


_dot_call = pl.kernel(
    _dot_body,
    out_type=(jax.ShapeDtypeStruct((EPAD,), jnp.float32),
              jax.ShapeDtypeStruct((NC, ACC_R, OUT), jnp.float32)),
    mesh=_mesh,
    compiler_params=_params,
    scratch_types=[
        pltpu.VMEM((DGRP, CH), jnp.int32),
        pltpu.VMEM((DGRP, CH), jnp.int32),
        pltpu.VMEM((DGRP * CH, OUT), jnp.float32),
        pltpu.VMEM((DGRP * CH, OUT), jnp.float32),
        pltpu.VMEM((DGRP, CH), jnp.int32),
        pltpu.VMEM((DGRP, CH), jnp.int32),
        pltpu.VMEM((DGRP * CH, OUT), jnp.float32),
        pltpu.VMEM((DGRP * CH, OUT), jnp.float32),
        pltpu.VMEM((DGRP * CH,), jnp.float32),
        pltpu.VMEM((CB, OUT), jnp.float32),
        pltpu.VMEM((CB, OUT), jnp.float32),
        pltpu.VMEM((CB, OUT), jnp.float32),
        pltpu.VMEM((CB, OUT), jnp.float32),
        pltpu.VMEM((CB, OUT), jnp.float32),
        pltpu.VMEM((CB, OUT), jnp.float32),
        pltpu.VMEM((CB, OUT), jnp.float32),
        pltpu.VMEM((CB, OUT), jnp.float32),
        pltpu.VMEM((CB, OUT), jnp.float32),
        pltpu.VMEM((CB, OUT), jnp.float32),
        pltpu.SemaphoreType.DMA,
        pltpu.SemaphoreType.DMA,
        pltpu.SemaphoreType.DMA,
        pltpu.SemaphoreType.DMA,
        pltpu.SemaphoreType.DMA,
        pltpu.SemaphoreType.DMA,
    ],
)


BN = 2000


def _prep_body(emb, W1, b1, W2, b2, d0, d1, z0_o, c_o, d2x_o, dfx_o, ch_o):
    x1 = jnp.maximum(
        jnp.dot(emb[...], W1[...], preferred_element_type=jnp.float32)
        + b1[...], 0.0)
    x0 = jnp.dot(x1, W2[...], preferred_element_type=jnp.float32) + b2[...]
    deg = d0[...] + d1[...] + 1.0
    dis = lax.rsqrt(deg)
    z0_o[...] = x0 * dis
    c_o[...] = ALPHA * dis * x0
    d2x_o[...] = jnp.broadcast_to((1.0 - ALPHA) * dis * dis, (BN, OUT))
    dfx_o[...] = jnp.broadcast_to((1.0 - ALPHA) * dis, (BN, OUT))
    ch_o[...] = ALPHA * x0


_prep_call = pl.pallas_call(
    _prep_body,
    grid=(N // BN,),
    in_specs=[
        pl.BlockSpec((BN, EMB), lambda i: (i, 0)),
        pl.BlockSpec((EMB, HID), lambda i: (0, 0)),
        pl.BlockSpec((1, HID), lambda i: (0, 0)),
        pl.BlockSpec((HID, OUT), lambda i: (0, 0)),
        pl.BlockSpec((1, OUT), lambda i: (0, 0)),
        pl.BlockSpec((BN, 1), lambda i: (i, 0)),
        pl.BlockSpec((BN, 1), lambda i: (i, 0)),
    ],
    out_specs=[pl.BlockSpec((BN, OUT), lambda i: (i, 0))] * 5,
    out_shape=[jax.ShapeDtypeStruct((ACC_R, OUT), jnp.float32)] * 5,
)


def kernel(edge_index, emb, W1, b1, W2, b2):
    src = edge_index[0]
    dst = edge_index[1]
    pad = EPAD - E
    srcp = jnp.concatenate(
        [src, jnp.zeros((pad,), jnp.int32)]).reshape(EROWS, CH)
    dstp = jnp.concatenate(
        [dst, jnp.full((pad,), N, jnp.int32)]).reshape(EROWS, CH)
    dstp0 = jnp.concatenate(
        [dst, jnp.zeros((pad,), jnp.int32)]).reshape(EROWS, CH)

    zeros1 = jnp.zeros((DZBLK,), jnp.float32)
    zeros2 = jnp.zeros((ZBLK, OUT), jnp.float32)

    dparts = _deg_call(dstp, zeros1)
    d0 = dparts[0].reshape(DEG_R, 1)
    d1 = dparts[1].reshape(DEG_R, 1)

    z0, cadd, d2x, dfx, ch = _prep_call(
        emb, W1, b1.reshape(1, HID), W2, b2.reshape(1, OUT), d0, d1)

    S = _scat_call(srcp, dstp, z0, zeros2)
    zpair = jnp.stack([z0, z0])
    for _ in range(1, K):
        S, zpair = _step_call(srcp, dstp, S, zpair, d2x, cadd, zeros2)

    pred, _ = _dot_call(srcp, dstp0, S, zpair, dfx, ch)
    return pred[:E]
